# baseline (speedup 1.0000x reference)
import functools

import jax
import jax.numpy as jnp
from jax import lax
from jax.experimental import pallas as pl
from jax.experimental.pallas import tpu as pltpu
from jax.experimental.pallas import tpu_sc as plsc

MOM = 0.99
N, C, H, W = 4, 96, 224, 224
HW = H * W
PER_CH = N * HW
NC, NS, L = 2, 16, 16
NW = NC * NS
CPT = C // NW
CH = HW // 7
CHW = CH // 4
NCHUNK = CPT * N * 7
GROUPS = CH // L
UNROLL = 8
BINS_T = CPT * 256
MAGIC = 8388608.0
PF = plsc.PackFormat.INTERLEAVED


def _flat_off(i, c0):
    ch_l = i // 28
    r = i % 28
    n = r // 7
    ck = r % 7
    row = n * C + c0 + ch_l
    return row * HW + ck * CH, ch_l


def _bin_off(i, c0):
    ch_l = i // 28
    r = i % 28
    n = r // 7
    ck = r % 7
    row = n * C + c0 + ch_l
    return row * (HW // 4) + ck * CHW


def _sc_body(x_hbm, ep_hbm, xste_hbm, px_hbm, bins_hbm,
             in0, in1, out0, out1, bu0, bu1, hist16, table, ep_v,
             si0, si1, so0, so1, sb0, sb1):
    wid = lax.axis_index("s") * NC + lax.axis_index("c")
    c0 = wid * CPT

    lane = lax.iota(jnp.int32, 16)
    lane768 = lane * BINS_T
    ones = jnp.full((16,), 1.0, jnp.float32)
    zeros = jnp.zeros((16,), jnp.float32)

    def zbody(g, _):
        hist16[pl.ds(g * 16, 16)] = zeros
    lax.fori_loop(0, L * BINS_T // 16, zbody, None)

    def start_in(src_hbm, i, buf, sem):
        off, _ = _flat_off(i, c0)
        pltpu.async_copy(src_hbm.at[pl.ds(off, CH)], buf, sem)

    def wait_in(src_hbm, buf, sem):
        pltpu.make_async_copy(src_hbm.at[pl.ds(0, CH)], buf, sem).wait()

    def start_out(dst_hbm, i, buf, sem):
        off, _ = _flat_off(i, c0)
        pltpu.async_copy(buf, dst_hbm.at[pl.ds(off, CH)], sem)

    def wait_out(dst_hbm, buf, sem):
        pltpu.make_async_copy(buf, dst_hbm.at[pl.ds(0, CH)], sem).wait()

    def start_bin_in(i, buf, sem):
        pltpu.async_copy(bins_hbm.at[pl.ds(_bin_off(i, c0), CHW)], buf, sem)

    def wait_bin_in(buf, sem):
        pltpu.make_async_copy(bins_hbm.at[pl.ds(0, CHW)], buf, sem).wait()

    def start_bin_out(i, buf, sem):
        pltpu.async_copy(buf, bins_hbm.at[pl.ds(_bin_off(i, c0), CHW)], sem)

    def wait_bin_out(buf, sem):
        pltpu.make_async_copy(buf, bins_hbm.at[pl.ds(0, CHW)], sem).wait()

    def p1_compute(i, ibuf, obuf, bbuf):
        _, ch_l = _flat_off(i, c0)
        base = lane768 + ch_l * 256

        def body(u, _):
            for q in range(UNROLL // 4):
                bs = []
                for k in range(4):
                    g = u * UNROLL + q * 4 + k
                    v = ibuf[pl.ds(g * 16, 16)]
                    v = jnp.minimum(v, 255.0)
                    rv = (v + MAGIC) - MAGIC
                    obuf[pl.ds(g * 16, 16)] = rv
                    b = rv.astype(jnp.int32)
                    plsc.addupdate_scatter(hist16, [b + base], ones)
                    bs.append(plsc.bitcast(b, jnp.uint32))
                h0 = plsc.pack(bs[0], bs[1], format=PF,
                               preferred_element_type=jnp.uint16)
                h1 = plsc.pack(bs[2], bs[3], format=PF,
                               preferred_element_type=jnp.uint16)
                packed = plsc.pack(h0, h1, format=PF,
                                   preferred_element_type=jnp.uint8)
                bbuf[pl.ds((u * UNROLL + q * 4) * 4, 16)] = plsc.bitcast(
                    packed, jnp.int32)
            return None
        lax.fori_loop(0, GROUPS // UNROLL, body, None)

    start_in(x_hbm, 0, in0, si0)
    start_in(x_hbm, 1, in1, si1)

    def p1_iter(j, _):
        ia = 2 * j
        ib = ia + 1
        wait_in(x_hbm, in0, si0)
        pl.when(j > 0)(lambda: wait_out(xste_hbm, out0, so0))
        pl.when(j > 0)(lambda: wait_bin_out(bu0, sb0))
        p1_compute(ia, in0, out0, bu0)
        pl.when(ia + 2 < NCHUNK)(lambda: start_in(x_hbm, ia + 2, in0, si0))
        start_out(xste_hbm, ia, out0, so0)
        start_bin_out(ia, bu0, sb0)

        wait_in(x_hbm, in1, si1)
        pl.when(j > 0)(lambda: wait_out(xste_hbm, out1, so1))
        pl.when(j > 0)(lambda: wait_bin_out(bu1, sb1))
        p1_compute(ib, in1, out1, bu1)
        pl.when(ib + 2 < NCHUNK)(lambda: start_in(x_hbm, ib + 2, in1, si1))
        start_out(xste_hbm, ib, out1, so1)
        start_bin_out(ib, bu1, sb1)
        return None

    lax.fori_loop(0, NCHUNK // 2, p1_iter, None)
    wait_out(xste_hbm, out0, so0)
    wait_out(xste_hbm, out1, so1)
    wait_bin_out(bu0, sb0)
    wait_bin_out(bu1, sb1)

    pltpu.sync_copy(ep_hbm.at[pl.ds(c0 * 256, BINS_T)], ep_v)

    def ema_body(g, _):
        acc = zeros
        for l in range(L):
            acc = acc + hist16[pl.ds(l * BINS_T + g * 16, 16)]
        e = ep_v[pl.ds(g * 16, 16)]
        table[pl.ds(g * 16, 16)] = e * MOM + acc * ((1.0 - MOM) / PER_CH)
        return None
    lax.fori_loop(0, BINS_T // 16, ema_body, None)

    def p2_compute(i, bbuf, obuf):
        _, ch_l = _flat_off(i, c0)
        toff = ch_l * 256

        def body(u, _):
            for q in range(UNROLL // 4):
                g0 = u * UNROLL + q * 4
                packed = plsc.bitcast(bbuf[pl.ds(g0 * 4, 16)], jnp.uint8)
                h0, h1 = plsc.unpack(packed, format=PF,
                                     preferred_element_type=jnp.uint16)
                b0, b1 = plsc.unpack(h0, format=PF,
                                     preferred_element_type=jnp.uint32)
                b2, b3 = plsc.unpack(h1, format=PF,
                                     preferred_element_type=jnp.uint32)
                for k, b in enumerate((b0, b1, b2, b3)):
                    idx = (plsc.bitcast(b, jnp.int32) & 255) + toff
                    obuf[pl.ds((g0 + k) * 16, 16)] = plsc.load_gather(table, [idx])
            return None
        lax.fori_loop(0, GROUPS // UNROLL, body, None)

    start_bin_in(0, bu0, si0)
    start_bin_in(1, bu1, si1)

    def p2_iter(j, _):
        ia = 2 * j
        ib = ia + 1
        wait_bin_in(bu0, si0)
        pl.when(j > 0)(lambda: wait_out(px_hbm, out0, so0))
        p2_compute(ia, bu0, out0)
        pl.when(ia + 2 < NCHUNK)(lambda: start_bin_in(ia + 2, bu0, si0))
        start_out(px_hbm, ia, out0, so0)

        wait_bin_in(bu1, si1)
        pl.when(j > 0)(lambda: wait_out(px_hbm, out1, so1))
        p2_compute(ib, bu1, out1)
        pl.when(ib + 2 < NCHUNK)(lambda: start_bin_in(ib + 2, bu1, si1))
        start_out(px_hbm, ib, out1, so1)
        return None

    lax.fori_loop(0, NCHUNK // 2, p2_iter, None)
    wait_out(px_hbm, out0, so0)
    wait_out(px_hbm, out1, so1)


@jax.jit
def kernel(x, estimated_p):
    total = N * C * HW
    xf = x.reshape(total)
    epf = estimated_p.reshape(C * 256)

    mesh = plsc.VectorSubcoreMesh(core_axis_name="c", subcore_axis_name="s")
    run = functools.partial(
        pl.kernel,
        out_type=[
            jax.ShapeDtypeStruct((total,), jnp.float32),
            jax.ShapeDtypeStruct((total,), jnp.float32),
            jax.ShapeDtypeStruct((total // 4,), jnp.int32),
        ],
        mesh=mesh,
        compiler_params=pltpu.CompilerParams(needs_layout_passes=False),
        scratch_types=[
            pltpu.VMEM((CH,), jnp.float32),
            pltpu.VMEM((CH,), jnp.float32),
            pltpu.VMEM((CH,), jnp.float32),
            pltpu.VMEM((CH,), jnp.float32),
            pltpu.VMEM((CHW,), jnp.int32),
            pltpu.VMEM((CHW,), jnp.int32),
            pltpu.VMEM((L * BINS_T,), jnp.float32),
            pltpu.VMEM((BINS_T,), jnp.float32),
            pltpu.VMEM((BINS_T,), jnp.float32),
            pltpu.SemaphoreType.DMA,
            pltpu.SemaphoreType.DMA,
            pltpu.SemaphoreType.DMA,
            pltpu.SemaphoreType.DMA,
            pltpu.SemaphoreType.DMA,
            pltpu.SemaphoreType.DMA,
        ],
    )(_sc_body)
    xste, px, _ = run(xf, epf)
    shape = (N, C, H, W)
    return xste.reshape(shape), px.reshape(shape)


# speedup vs baseline: 1.3962x; 1.3962x over previous
import functools

import jax
import jax.numpy as jnp
from jax import lax
from jax.experimental import pallas as pl
from jax.experimental.pallas import tpu as pltpu
from jax.experimental.pallas import tpu_sc as plsc

MOM = 0.99
N, C, H, W = 4, 96, 224, 224
HW = H * W
PER_CH = N * HW
NC, NS, L = 2, 16, 16
NW = NC * NS
CPT = C // NW
RC = H // 4
CH = RC * W
CHW = CH // 4
NCHUNK = CPT * N * 4
RP = RC // 2
BINS_T = CPT * 256
MAGIC = 8388608.0
PF = plsc.PackFormat.INTERLEAVED

_GOFF = [((64 * q + 16 * k) // W, (64 * q + 16 * k) % W)
         for q in range(7) for k in range(4)]


def _chunk_id(i, c0):
    ch_l = i >> 4
    r = i & 15
    n = r >> 2
    ck = r & 3
    return n, c0 + ch_l, ck, ch_l


def _bin_off(i, c0):
    n, c, ck, _ = _chunk_id(i, c0)
    return (n * C + c) * (HW // 4) + ck * CHW


def _sc_body(x_hbm, ep_hbm, xste_hbm, px_hbm, bins_hbm,
             in0, in1, out0, out1, bu0, bu1, hist16, table, ep_v,
             si0, si1, so0, so1, sb0, sb1):
    wid = lax.axis_index("s") * NC + lax.axis_index("c")
    c0 = wid * CPT

    lane = lax.iota(jnp.int32, 16)
    lane768 = lane * BINS_T
    ones = jnp.full((16,), 1.0, jnp.float32)
    zeros = jnp.zeros((16,), jnp.float32)

    def zbody(g, _):
        hist16[pl.ds(g * 16, 16)] = zeros
    lax.fori_loop(0, 16 * BINS_T // 16, zbody, None)

    def img_slice(hbm, i):
        n, c, ck, _ = _chunk_id(i, c0)
        return hbm.at[n, c, pl.ds(ck * RC, RC), :]

    def start_in(src_hbm, i, buf, sem):
        pltpu.async_copy(img_slice(src_hbm, i), buf, sem)

    def wait_img(hbm_ref, buf, sem):
        pltpu.make_async_copy(hbm_ref.at[0, 0, pl.ds(0, RC), :], buf, sem).wait()

    def start_out(dst_hbm, i, buf, sem):
        pltpu.async_copy(buf, img_slice(dst_hbm, i), sem)

    def start_bin_in(i, buf, sem):
        pltpu.async_copy(bins_hbm.at[pl.ds(_bin_off(i, c0), CHW)], buf, sem)

    def wait_bin(buf, sem):
        pltpu.make_async_copy(bins_hbm.at[pl.ds(0, CHW)], buf, sem).wait()

    def start_bin_out(i, buf, sem):
        pltpu.async_copy(buf, bins_hbm.at[pl.ds(_bin_off(i, c0), CHW)], sem)

    def p1_compute(i, ibuf, obuf, bbuf):
        _, _, _, ch_l = _chunk_id(i, c0)
        base = lane768 + ch_l * 256

        def body(rr, _):
            r0 = rr * 2
            for q in range(7):
                bs = []
                for k in range(4):
                    dr, dc = _GOFF[q * 4 + k]
                    v = ibuf[r0 + dr, pl.ds(dc, 16)]
                    v = jnp.minimum(v, 255.0)
                    rv = (v + MAGIC) - MAGIC
                    obuf[r0 + dr, pl.ds(dc, 16)] = rv
                    b = rv.astype(jnp.int32)
                    plsc.addupdate_scatter(hist16, [b + base], ones)
                    bs.append(plsc.bitcast(b, jnp.uint32))
                h0 = plsc.pack(bs[0], bs[1], format=PF,
                               preferred_element_type=jnp.uint16)
                h1 = plsc.pack(bs[2], bs[3], format=PF,
                               preferred_element_type=jnp.uint16)
                packed = plsc.pack(h0, h1, format=PF,
                                   preferred_element_type=jnp.uint8)
                bbuf[pl.ds(rr * 112 + q * 16, 16)] = plsc.bitcast(
                    packed, jnp.int32)
            return None
        lax.fori_loop(0, RP, body, None)

    start_in(x_hbm, 0, in0, si0)
    start_in(x_hbm, 1, in1, si1)

    def p1_iter(j, _):
        ia = 2 * j
        ib = ia + 1
        wait_img(x_hbm, in0, si0)
        pl.when(j > 0)(lambda: wait_img(xste_hbm, out0, so0))
        pl.when(j > 0)(lambda: wait_bin(bu0, sb0))
        p1_compute(ia, in0, out0, bu0)
        pl.when(ia + 2 < NCHUNK)(lambda: start_in(x_hbm, ia + 2, in0, si0))
        start_out(xste_hbm, ia, out0, so0)
        start_bin_out(ia, bu0, sb0)

        wait_img(x_hbm, in1, si1)
        pl.when(j > 0)(lambda: wait_img(xste_hbm, out1, so1))
        pl.when(j > 0)(lambda: wait_bin(bu1, sb1))
        p1_compute(ib, in1, out1, bu1)
        pl.when(ib + 2 < NCHUNK)(lambda: start_in(x_hbm, ib + 2, in1, si1))
        start_out(xste_hbm, ib, out1, so1)
        start_bin_out(ib, bu1, sb1)
        return None

    lax.fori_loop(0, NCHUNK // 2, p1_iter, None)
    wait_img(xste_hbm, out0, so0)
    wait_img(xste_hbm, out1, so1)
    wait_bin(bu0, sb0)
    wait_bin(bu1, sb1)

    pltpu.sync_copy(ep_hbm, ep_v)

    def ema_body(g, _):
        acc = zeros
        for l in range(16):
            acc = acc + hist16[pl.ds(l * BINS_T + g * 16, 16)]
        ch_l = g // 16
        e = ep_v[c0 + ch_l, pl.ds((g % 16) * 16, 16)]
        table[pl.ds(g * 16, 16)] = e * MOM + acc * ((1.0 - MOM) / PER_CH)
        return None
    lax.fori_loop(0, BINS_T // 16, ema_body, None)

    def p2_compute(i, bbuf, obuf):
        _, _, _, ch_l = _chunk_id(i, c0)
        toff = ch_l * 256

        def body(rr, _):
            r0 = rr * 2
            for q in range(7):
                packed = plsc.bitcast(bbuf[pl.ds(rr * 112 + q * 16, 16)],
                                      jnp.uint8)
                h0, h1 = plsc.unpack(packed, format=PF,
                                     preferred_element_type=jnp.uint16)
                b0, b1 = plsc.unpack(h0, format=PF,
                                     preferred_element_type=jnp.uint32)
                b2, b3 = plsc.unpack(h1, format=PF,
                                     preferred_element_type=jnp.uint32)
                for k, b in enumerate((b0, b1, b2, b3)):
                    dr, dc = _GOFF[q * 4 + k]
                    idx = (plsc.bitcast(b, jnp.int32) & 255) + toff
                    obuf[r0 + dr, pl.ds(dc, 16)] = plsc.load_gather(table, [idx])
            return None
        lax.fori_loop(0, RP, body, None)

    start_bin_in(0, bu0, si0)
    start_bin_in(1, bu1, si1)

    def p2_iter(j, _):
        ia = 2 * j
        ib = ia + 1
        wait_bin(bu0, si0)
        pl.when(j > 0)(lambda: wait_img(px_hbm, out0, so0))
        p2_compute(ia, bu0, out0)
        pl.when(ia + 2 < NCHUNK)(lambda: start_bin_in(ia + 2, bu0, si0))
        start_out(px_hbm, ia, out0, so0)

        wait_bin(bu1, si1)
        pl.when(j > 0)(lambda: wait_img(px_hbm, out1, so1))
        p2_compute(ib, bu1, out1)
        pl.when(ib + 2 < NCHUNK)(lambda: start_bin_in(ib + 2, bu1, si1))
        start_out(px_hbm, ib, out1, so1)
        return None

    lax.fori_loop(0, NCHUNK // 2, p2_iter, None)
    wait_img(px_hbm, out0, so0)
    wait_img(px_hbm, out1, so1)


@jax.jit
def kernel(x, estimated_p):
    total = N * C * HW
    mesh = plsc.VectorSubcoreMesh(core_axis_name="c", subcore_axis_name="s")
    run = functools.partial(
        pl.kernel,
        out_type=[
            jax.ShapeDtypeStruct((N, C, H, W), jnp.float32),
            jax.ShapeDtypeStruct((N, C, H, W), jnp.float32),
            jax.ShapeDtypeStruct((total // 4,), jnp.int32),
        ],
        mesh=mesh,
        compiler_params=pltpu.CompilerParams(needs_layout_passes=False),
        scratch_types=[
            pltpu.VMEM((RC, W), jnp.float32),
            pltpu.VMEM((RC, W), jnp.float32),
            pltpu.VMEM((RC, W), jnp.float32),
            pltpu.VMEM((RC, W), jnp.float32),
            pltpu.VMEM((CHW,), jnp.int32),
            pltpu.VMEM((CHW,), jnp.int32),
            pltpu.VMEM((16 * BINS_T,), jnp.float32),
            pltpu.VMEM((BINS_T,), jnp.float32),
            pltpu.VMEM((C, 256), jnp.float32),
            pltpu.SemaphoreType.DMA,
            pltpu.SemaphoreType.DMA,
            pltpu.SemaphoreType.DMA,
            pltpu.SemaphoreType.DMA,
            pltpu.SemaphoreType.DMA,
            pltpu.SemaphoreType.DMA,
        ],
    )(_sc_body)
    xste, px, _ = run(x, estimated_p)
    return xste, px
